# baseline (speedup 1.0000x reference)
import functools

import jax
import jax.numpy as jnp
import numpy as np
from jax.experimental import pallas as pl
from jax.experimental.pallas import tpu as pltpu


def _s2d_pad(h):
    B, H, W, C = h.shape
    xp = jnp.pad(h, ((0, 0), (2, 2), (2, 2), (0, 0)))
    Hs, Ws = (H + 4) // 2, (W + 4) // 2
    xs = xp.reshape(B, Hs, 2, Ws, 2, C).transpose(0, 1, 3, 2, 4, 5)
    return xs.reshape(B, Hs, Ws, 4 * C).astype(jnp.bfloat16)


def _w_s2d(w):
    C, O = w.shape[2], w.shape[3]
    wp = jnp.pad(w, ((0, 1), (0, 1), (0, 0), (0, 0)))
    wt = wp.reshape(3, 2, 3, 2, C, O).transpose(0, 2, 1, 3, 4, 5)
    return wt.reshape(3, 3, 4 * C, O)


def _tap_pieces(C):
    pieces = []
    for a in range(3):
        for b in range(3):
            if a < 2 and b < 2:
                pieces.append((a, b, 0, 4 * C))
            elif a == 2 and b < 2:
                pieces.append((a, b, 0, 2 * C))
            elif a < 2 and b == 2:
                pieces.append((a, b, 0, C))
                pieces.append((a, b, 2 * C, 3 * C))
            else:
                pieces.append((a, b, 0, C))
    return pieces


def _pack_weight(w):
    C = w.shape[2]
    wt = _w_s2d(w)
    parts = [wt[a, b, c0:c1, :] for (a, b, c0, c1) in _tap_pieces(C)]
    return jnp.concatenate(parts, axis=0).astype(jnp.bfloat16)


def _gemm_bias_relu_kernel(p_ref, w_ref, b_ref, o_ref):
    acc = jnp.dot(p_ref[...], w_ref[...], preferred_element_type=jnp.float32)
    o_ref[...] = jnp.maximum(acc + b_ref[...], 0.0).astype(o_ref.dtype)


def _conv_s2d_kernel(Ho, Wo, C, x_ref, w_ref, b_ref, o_ref):
    x = x_ref[...]
    k = x.shape[0]
    parts = [
        x[:, a:a + Ho, b:b + Wo, c0:c1].reshape(k * Ho * Wo, c1 - c0)
        for (a, b, c0, c1) in _tap_pieces(C)
    ]
    p = jnp.concatenate(parts, axis=-1)
    acc = jnp.dot(p, w_ref[...], preferred_element_type=jnp.float32)
    y = jnp.maximum(acc + b_ref[...], 0.0)
    o_ref[...] = y.reshape(k, Ho, Wo, -1).astype(o_ref.dtype)


def _conv_head_kernel(Ho, Wo, C, x_ref, w_ref, b_ref, w5_ref, b5_ref, o_ref):
    x = x_ref[...]
    k = x.shape[0]
    parts = [
        x[:, a:a + Ho, b:b + Wo, c0:c1].reshape(k * Ho * Wo, c1 - c0)
        for (a, b, c0, c1) in _tap_pieces(C)
    ]
    p = jnp.concatenate(parts, axis=-1)
    acc = jnp.dot(p, w_ref[...], preferred_element_type=jnp.float32)
    h = jnp.maximum(acc + b_ref[...], 0.0)
    hb = h.reshape(k, Ho * Wo, -1)
    logits = jnp.sum(hb * w5_ref[...][None], axis=(1, 2)) + b5_ref[0, 0]
    o_ref[...] = jax.nn.sigmoid(logits).reshape(1, k, 1)


def _params(vmem_mb):
    return pltpu.CompilerParams(
        dimension_semantics=("parallel",),
        vmem_limit_bytes=vmem_mb << 20,
    )


def _conv1(patches, w_mat, bias, TM):
    M, K = patches.shape
    N = w_mat.shape[1]
    grid = M // TM
    return pl.pallas_call(
        _gemm_bias_relu_kernel,
        out_shape=jax.ShapeDtypeStruct((M, N), jnp.bfloat16),
        grid=(grid,),
        in_specs=[
            pl.BlockSpec((TM, K), lambda i: (i, 0)),
            pl.BlockSpec((K, N), lambda i: (0, 0)),
            pl.BlockSpec((1, N), lambda i: (0, 0)),
        ],
        out_specs=pl.BlockSpec((TM, N), lambda i: (i, 0)),
        compiler_params=_params(50),
    )(patches, w_mat, bias)


def _conv_s2d(xs, w_mat, bias, k):
    B, Hs, Ws, C4 = xs.shape
    C = C4 // 4
    Ho, Wo = Hs - 2, Ws - 2
    K, N = w_mat.shape
    return pl.pallas_call(
        functools.partial(_conv_s2d_kernel, Ho, Wo, C),
        out_shape=jax.ShapeDtypeStruct((B, Ho, Wo, N), jnp.bfloat16),
        grid=(B // k,),
        in_specs=[
            pl.BlockSpec((k, Hs, Ws, C4), lambda i: (i, 0, 0, 0)),
            pl.BlockSpec((K, N), lambda i: (0, 0)),
            pl.BlockSpec((1, N), lambda i: (0, 0)),
        ],
        out_specs=pl.BlockSpec((k, Ho, Wo, N), lambda i: (i, 0, 0, 0)),
        compiler_params=_params(40),
    )(xs, w_mat, bias)


def _conv_head(xs, w_mat, bias, w5_mat, b5, k):
    B, Hs, Ws, C4 = xs.shape
    C = C4 // 4
    Ho, Wo = Hs - 2, Ws - 2
    K, N = w_mat.shape
    out = pl.pallas_call(
        functools.partial(_conv_head_kernel, Ho, Wo, C),
        out_shape=jax.ShapeDtypeStruct((B // k, k, 1), jnp.float32),
        grid=(B // k,),
        in_specs=[
            pl.BlockSpec((k, Hs, Ws, C4), lambda i: (i, 0, 0, 0)),
            pl.BlockSpec((K, N), lambda i: (0, 0)),
            pl.BlockSpec((1, N), lambda i: (0, 0)),
            pl.BlockSpec(w5_mat.shape, lambda i: (0, 0)),
            pl.BlockSpec((1, 1), lambda i: (0, 0)),
        ],
        out_specs=pl.BlockSpec((1, k, 1), lambda i: (i, 0, 0)),
        compiler_params=_params(44),
    )(xs, w_mat, bias, w5_mat, b5)
    return out.reshape(B, 1)


def kernel(x, w1, b1, w2, b2, w3, b3, w4, b4, w5, b5):
    B = x.shape[0]
    H = x.shape[2]
    Ho = H // 2

    xh = jnp.transpose(x, (0, 2, 3, 1))
    xf = jnp.pad(xh, ((0, 0), (2, 2), (2, 6), (0, 0))).astype(jnp.bfloat16)
    xfv = xf.reshape(B, H + 4, -1)
    xv = jnp.stack([xfv[:, kh:kh + 2 * Ho:2, :] for kh in range(5)], axis=2)
    p1 = xv.reshape(B * Ho, 5 * 408)

    n1 = w1.shape[3]
    sel = np.zeros((17, 8, Ho, 5), np.float32)
    for ow in range(Ho):
        for kw in range(5):
            g, wp = divmod(2 * ow + kw, 8)
            sel[g, wp, ow, kw] = 1.0
    w1big = jnp.einsum("uwok,hkcn->huwcon", jnp.asarray(sel), w1)
    w1m = w1big.reshape(2040, Ho * n1).astype(jnp.bfloat16)
    b1t = jnp.tile(b1, (1, Ho))
    a1 = _conv1(p1, w1m, b1t, TM=512)
    h = a1.reshape(B, Ho, Ho, n1)

    h = _conv_s2d(_s2d_pad(h), _pack_weight(w2), b2, k=4)
    h = _conv_s2d(_s2d_pad(h), _pack_weight(w3), b3, k=4)

    w5m = w5[:, 0].reshape(64, -1)
    return _conv_head(_s2d_pad(h), _pack_weight(w4), b4, w5m, b5, k=8)


# speedup vs baseline: 5.2240x; 5.2240x over previous
import functools

import jax
import jax.numpy as jnp
import numpy as np
from jax.experimental import pallas as pl
from jax.experimental.pallas import tpu as pltpu


def _s2d_pad(h):
    B, H, W, C = h.shape
    xp = jnp.pad(h, ((0, 0), (2, 2), (2, 2), (0, 0)))
    Hs, Ws = (H + 4) // 2, (W + 4) // 2
    xs = xp.reshape(B, Hs, 2, Ws, 2, C).transpose(0, 1, 3, 2, 4, 5)
    return xs.reshape(B, Hs, Ws, 4 * C).astype(jnp.bfloat16)


def _w_s2d(w):
    C, O = w.shape[2], w.shape[3]
    wp = jnp.pad(w, ((0, 1), (0, 1), (0, 0), (0, 0)))
    wt = wp.reshape(3, 2, 3, 2, C, O).transpose(0, 2, 1, 3, 4, 5)
    return wt.reshape(3, 3, 4 * C, O)


def _tap_pieces(C):
    pieces = []
    for a in range(3):
        for b in range(3):
            if a < 2 and b < 2:
                pieces.append((a, b, 0, 4 * C))
            elif a == 2 and b < 2:
                pieces.append((a, b, 0, 2 * C))
            elif a < 2 and b == 2:
                pieces.append((a, b, 0, C))
                pieces.append((a, b, 2 * C, 3 * C))
            else:
                pieces.append((a, b, 0, C))
    return pieces


def _pack_weight(w):
    C = w.shape[2]
    wt = _w_s2d(w)
    parts = [wt[a, b, c0:c1, :] for (a, b, c0, c1) in _tap_pieces(C)]
    return jnp.concatenate(parts, axis=0).astype(jnp.bfloat16)


def _gemm_bias_relu_kernel(p_ref, w_ref, b_ref, o_ref):
    acc = jnp.dot(p_ref[...], w_ref[...], preferred_element_type=jnp.float32)
    o_ref[...] = jnp.maximum(acc + b_ref[...], 0.0).astype(o_ref.dtype)


def _conv1_win_kernel(x_ref, w_ref, b_ref, o_ref):
    x = x_ref[...]
    k = x.shape[0]
    parts = [x[:, kh].reshape(k * 64 * 16, 48) for kh in range(5)]
    p = jnp.concatenate(parts, axis=-1)
    acc = jnp.dot(p, w_ref[...], preferred_element_type=jnp.float32)
    y = jnp.maximum(acc + b_ref[...], 0.0)
    o_ref[...] = y.reshape(k, 64, 16, -1).astype(o_ref.dtype)


def _conv_s2d_kernel(Ho, Wo, C, x_ref, w_ref, b_ref, o_ref):
    x = x_ref[...]
    k = x.shape[0]
    parts = [
        x[:, a:a + Ho, b:b + Wo, c0:c1].reshape(k * Ho * Wo, c1 - c0)
        for (a, b, c0, c1) in _tap_pieces(C)
    ]
    p = jnp.concatenate(parts, axis=-1)
    acc = jnp.dot(p, w_ref[...], preferred_element_type=jnp.float32)
    y = jnp.maximum(acc + b_ref[...], 0.0)
    o_ref[...] = y.reshape(k, Ho, Wo, -1).astype(o_ref.dtype)


def _conv_head_kernel(Ho, Wo, C, x_ref, w_ref, b_ref, w5_ref, b5_ref, o_ref):
    x = x_ref[...]
    k = x.shape[0]
    parts = [
        x[:, a:a + Ho, b:b + Wo, c0:c1].reshape(k * Ho * Wo, c1 - c0)
        for (a, b, c0, c1) in _tap_pieces(C)
    ]
    p = jnp.concatenate(parts, axis=-1)
    acc = jnp.dot(p, w_ref[...], preferred_element_type=jnp.float32)
    h = jnp.maximum(acc + b_ref[...], 0.0)
    hb = h.reshape(k, Ho * Wo, -1)
    logits = jnp.sum(hb * w5_ref[...][None], axis=(1, 2)) + b5_ref[0, 0]
    o_ref[...] = jax.nn.sigmoid(logits).reshape(1, k, 1)


def _params(vmem_mb):
    return pltpu.CompilerParams(
        dimension_semantics=("parallel",),
        vmem_limit_bytes=vmem_mb << 20,
    )


def _conv1(patches, w_mat, bias, TM):
    M, K = patches.shape
    N = w_mat.shape[1]
    grid = M // TM
    return pl.pallas_call(
        _gemm_bias_relu_kernel,
        out_shape=jax.ShapeDtypeStruct((M, N), jnp.bfloat16),
        grid=(grid,),
        in_specs=[
            pl.BlockSpec((TM, K), lambda i: (i, 0)),
            pl.BlockSpec((K, N), lambda i: (0, 0)),
            pl.BlockSpec((1, N), lambda i: (0, 0)),
        ],
        out_specs=pl.BlockSpec((TM, N), lambda i: (i, 0)),
        compiler_params=_params(50),
    )(patches, w_mat, bias)


def _conv1_win(xv, w_mat, bias, k):
    B = xv.shape[0]
    N = w_mat.shape[1]
    return pl.pallas_call(
        _conv1_win_kernel,
        out_shape=jax.ShapeDtypeStruct((B, 64, 16, N), jnp.bfloat16),
        grid=(B // k,),
        in_specs=[
            pl.BlockSpec((k, 5, 64, 16, 48), lambda i: (i, 0, 0, 0, 0)),
            pl.BlockSpec(w_mat.shape, lambda i: (0, 0)),
            pl.BlockSpec((1, N), lambda i: (0, 0)),
        ],
        out_specs=pl.BlockSpec((k, 64, 16, N), lambda i: (i, 0, 0, 0)),
        compiler_params=_params(32),
    )(xv, w_mat, bias)


def _conv_s2d(xs, w_mat, bias, k):
    B, Hs, Ws, C4 = xs.shape
    C = C4 // 4
    Ho, Wo = Hs - 2, Ws - 2
    K, N = w_mat.shape
    return pl.pallas_call(
        functools.partial(_conv_s2d_kernel, Ho, Wo, C),
        out_shape=jax.ShapeDtypeStruct((B, Ho, Wo, N), jnp.bfloat16),
        grid=(B // k,),
        in_specs=[
            pl.BlockSpec((k, Hs, Ws, C4), lambda i: (i, 0, 0, 0)),
            pl.BlockSpec((K, N), lambda i: (0, 0)),
            pl.BlockSpec((1, N), lambda i: (0, 0)),
        ],
        out_specs=pl.BlockSpec((k, Ho, Wo, N), lambda i: (i, 0, 0, 0)),
        compiler_params=_params(40),
    )(xs, w_mat, bias)


def _conv_head(xs, w_mat, bias, w5_mat, b5, k):
    B, Hs, Ws, C4 = xs.shape
    C = C4 // 4
    Ho, Wo = Hs - 2, Ws - 2
    K, N = w_mat.shape
    out = pl.pallas_call(
        functools.partial(_conv_head_kernel, Ho, Wo, C),
        out_shape=jax.ShapeDtypeStruct((B // k, k, 1), jnp.float32),
        grid=(B // k,),
        in_specs=[
            pl.BlockSpec((k, Hs, Ws, C4), lambda i: (i, 0, 0, 0)),
            pl.BlockSpec((K, N), lambda i: (0, 0)),
            pl.BlockSpec((1, N), lambda i: (0, 0)),
            pl.BlockSpec(w5_mat.shape, lambda i: (0, 0)),
            pl.BlockSpec((1, 1), lambda i: (0, 0)),
        ],
        out_specs=pl.BlockSpec((1, k, 1), lambda i: (i, 0, 0)),
        compiler_params=_params(44),
    )(xs, w_mat, bias, w5_mat, b5)
    return out.reshape(B, 1)


def kernel(x, w1, b1, w2, b2, w3, b3, w4, b4, w5, b5):
    B = x.shape[0]
    H = x.shape[2]
    Ho = H // 2

    xh = jnp.transpose(x, (0, 2, 3, 1))
    xf = jnp.pad(xh, ((0, 0), (2, 2), (2, 6), (0, 0))).astype(jnp.bfloat16)
    xfg = xf.reshape(B, H + 4, 17, 24)
    pc = jnp.concatenate([xfg[:, :, :16, :], xfg[:, :, 1:, :]], axis=-1)
    xv = jnp.stack([pc[:, kh:kh + 2 * Ho:2] for kh in range(5)], axis=1)

    n1 = w1.shape[3]
    sel = np.zeros((2, 8, 4, 5), np.float32)
    for s in range(4):
        for kw in range(5):
            g, wp = divmod(2 * s + kw, 8)
            sel[g, wp, s, kw] = 1.0
    w1q = jnp.einsum("gwsk,hkcn->hgwcsn", jnp.asarray(sel), w1)
    w1m = w1q.reshape(240, 4 * n1).astype(jnp.bfloat16)
    b1t = jnp.tile(b1, (1, 4))
    a1 = _conv1_win(xv, w1m, b1t, k=4)
    h = a1.reshape(B, Ho, Ho, n1)

    h = _conv_s2d(_s2d_pad(h), _pack_weight(w2), b2, k=4)
    h = _conv_s2d(_s2d_pad(h), _pack_weight(w3), b3, k=4)

    w5m = w5[:, 0].reshape(64, -1)
    return _conv_head(_s2d_pad(h), _pack_weight(w4), b4, w5m, b5, k=8)


# speedup vs baseline: 5.5927x; 1.0706x over previous
import functools

import jax
import jax.numpy as jnp
import numpy as np
from jax.experimental import pallas as pl
from jax.experimental.pallas import tpu as pltpu


def _pad_fold(h):
    B, H, W, C = h.shape
    xp = jnp.pad(h, ((0, 0), (2, 2), (2, 2), (0, 0)))
    return xp.reshape(B, H // 2 + 2, 2, (W + 4) // 2, 2 * C)


_FOLD_TAPS = ((0, 2), (1, 2), (2, 1))


def _pack_weight_fold(w):
    C, O = w.shape[2], w.shape[3]
    wp = jnp.pad(w, ((0, 0), (0, 1), (0, 0), (0, 0)))
    wf = wp.reshape(5, 3, 2, C, O)
    parts = []
    for kh in range(5):
        parts.append(wf[kh, 0].reshape(2 * C, O))
        parts.append(wf[kh, 1].reshape(2 * C, O))
        parts.append(wf[kh, 2, 0])
    return jnp.concatenate(parts, axis=0).astype(jnp.bfloat16)


def _conv1_win_kernel(x_ref, w_ref, b_ref, o_ref):
    x = x_ref[...]
    k = x.shape[0]
    parts = [x[:, kh].reshape(k * 64 * 16, 48) for kh in range(5)]
    p = jnp.concatenate(parts, axis=-1)
    acc = jnp.dot(p, w_ref[...], preferred_element_type=jnp.float32)
    y = jnp.maximum(acc + b_ref[...], 0.0)
    o_ref[...] = y.reshape(k, 64, 16, -1).astype(o_ref.dtype)


def _fold_patches(x, Ho, Wo, C):
    k = x.shape[0]
    parts = []
    for kh in range(5):
        a0, p0 = divmod(kh, 2)
        xk = x[:, a0:a0 + Ho, p0]
        for bw, units in _FOLD_TAPS:
            cw = units * C
            parts.append(xk[:, :, bw:bw + Wo, :cw].reshape(k * Ho * Wo, cw))
    return jnp.concatenate(parts, axis=-1)


def _conv_fold_kernel(Ho, Wo, C, x_ref, w_ref, b_ref, o_ref):
    x = x_ref[...]
    k = x.shape[0]
    p = _fold_patches(x, Ho, Wo, C)
    acc = jnp.dot(p, w_ref[...], preferred_element_type=jnp.float32)
    y = jnp.maximum(acc + b_ref[...], 0.0)
    o_ref[...] = y.reshape(k, Ho, Wo, -1).astype(o_ref.dtype)


def _conv_head_kernel(Ho, Wo, C, x_ref, w_ref, b_ref, w5_ref, b5_ref, o_ref):
    x = x_ref[...]
    k = x.shape[0]
    p = _fold_patches(x, Ho, Wo, C)
    acc = jnp.dot(p, w_ref[...], preferred_element_type=jnp.float32)
    h = jnp.maximum(acc + b_ref[...], 0.0)
    hb = h.reshape(k, Ho * Wo, -1)
    logits = jnp.sum(hb * w5_ref[...][None], axis=(1, 2)) + b5_ref[0, 0]
    o_ref[...] = jax.nn.sigmoid(logits).reshape(1, k, 1)


def _params(vmem_mb):
    return pltpu.CompilerParams(
        dimension_semantics=("parallel",),
        vmem_limit_bytes=vmem_mb << 20,
    )


def _conv1_win(xv, w_mat, bias, k):
    B = xv.shape[0]
    N = w_mat.shape[1]
    return pl.pallas_call(
        _conv1_win_kernel,
        out_shape=jax.ShapeDtypeStruct((B, 64, 16, N), jnp.bfloat16),
        grid=(B // k,),
        in_specs=[
            pl.BlockSpec((k, 5, 64, 16, 48), lambda i: (i, 0, 0, 0, 0)),
            pl.BlockSpec(w_mat.shape, lambda i: (0, 0)),
            pl.BlockSpec((1, N), lambda i: (0, 0)),
        ],
        out_specs=pl.BlockSpec((k, 64, 16, N), lambda i: (i, 0, 0, 0)),
        compiler_params=_params(32),
    )(xv, w_mat, bias)


def _fold_dims(xs):
    B, Ha, _, Wf, C2 = xs.shape
    return B, Ha - 2, Wf - 2, C2 // 2


def _conv_fold(xs, w_mat, bias, k):
    B, Ho, Wo, C = _fold_dims(xs)
    K, N = w_mat.shape
    return pl.pallas_call(
        functools.partial(_conv_fold_kernel, Ho, Wo, C),
        out_shape=jax.ShapeDtypeStruct((B, Ho, Wo, N), jnp.bfloat16),
        grid=(B // k,),
        in_specs=[
            pl.BlockSpec((k,) + xs.shape[1:], lambda i: (i, 0, 0, 0, 0)),
            pl.BlockSpec((K, N), lambda i: (0, 0)),
            pl.BlockSpec((1, N), lambda i: (0, 0)),
        ],
        out_specs=pl.BlockSpec((k, Ho, Wo, N), lambda i: (i, 0, 0, 0)),
        compiler_params=_params(40),
    )(xs, w_mat, bias)


def _conv_head(xs, w_mat, bias, w5_mat, b5, k):
    B, Ho, Wo, C = _fold_dims(xs)
    K, N = w_mat.shape
    out = pl.pallas_call(
        functools.partial(_conv_head_kernel, Ho, Wo, C),
        out_shape=jax.ShapeDtypeStruct((B // k, k, 1), jnp.float32),
        grid=(B // k,),
        in_specs=[
            pl.BlockSpec((k,) + xs.shape[1:], lambda i: (i, 0, 0, 0, 0)),
            pl.BlockSpec((K, N), lambda i: (0, 0)),
            pl.BlockSpec((1, N), lambda i: (0, 0)),
            pl.BlockSpec(w5_mat.shape, lambda i: (0, 0)),
            pl.BlockSpec((1, 1), lambda i: (0, 0)),
        ],
        out_specs=pl.BlockSpec((1, k, 1), lambda i: (i, 0, 0)),
        compiler_params=_params(44),
    )(xs, w_mat, bias, w5_mat, b5)
    return out.reshape(B, 1)


def kernel(x, w1, b1, w2, b2, w3, b3, w4, b4, w5, b5):
    B = x.shape[0]
    H = x.shape[2]
    Ho = H // 2

    xh = jnp.transpose(x, (0, 2, 3, 1))
    xf = jnp.pad(xh, ((0, 0), (2, 2), (2, 6), (0, 0))).astype(jnp.bfloat16)
    xfg = xf.reshape(B, H + 4, 17, 24)
    pc = jnp.concatenate([xfg[:, :, :16, :], xfg[:, :, 1:, :]], axis=-1)
    xv = jnp.stack([pc[:, kh:kh + 2 * Ho:2] for kh in range(5)], axis=1)

    n1 = w1.shape[3]
    sel = np.zeros((2, 8, 4, 5), np.float32)
    for s in range(4):
        for kw in range(5):
            g, wp = divmod(2 * s + kw, 8)
            sel[g, wp, s, kw] = 1.0
    w1q = jnp.einsum("gwsk,hkcn->hgwcsn", jnp.asarray(sel), w1)
    w1m = w1q.reshape(240, 4 * n1).astype(jnp.bfloat16)
    b1t = jnp.tile(b1, (1, 4))
    a1 = _conv1_win(xv, w1m, b1t, k=4)
    h = a1.reshape(B, Ho, Ho, n1)

    h = _conv_fold(_pad_fold(h), _pack_weight_fold(w2), b2, k=4)
    h = _conv_fold(_pad_fold(h), _pack_weight_fold(w3), b3, k=4)

    w5m = w5[:, 0].reshape(64, -1)
    return _conv_head(_pad_fold(h), _pack_weight_fold(w4), b4, w5m, b5, k=8)


# speedup vs baseline: 5.6262x; 1.0060x over previous
import functools

import jax
import jax.numpy as jnp
import numpy as np
from jax.experimental import pallas as pl
from jax.experimental.pallas import tpu as pltpu


def _pad_fold(h):
    B, H, W, C = h.shape
    xp = jnp.pad(h, ((0, 0), (2, 2), (2, 2), (0, 0)))
    return xp.reshape(B, H // 2 + 2, 2, (W + 4) // 2, 2 * C)


_FOLD_TAPS = ((0, 2), (1, 2), (2, 1))


def _pack_weight_fold(w):
    C, O = w.shape[2], w.shape[3]
    wp = jnp.pad(w, ((0, 0), (0, 1), (0, 0), (0, 0)))
    wf = wp.reshape(5, 3, 2, C, O)
    parts = []
    for kh in range(5):
        parts.append(wf[kh, 0].reshape(2 * C, O))
        parts.append(wf[kh, 1].reshape(2 * C, O))
        parts.append(wf[kh, 2, 0])
    return jnp.concatenate(parts, axis=0).astype(jnp.bfloat16)


def _conv1_win_kernel(x_ref, w_ref, b_ref, o_ref):
    x = x_ref[...]
    k = x.shape[0]
    parts = [x[:, kh].reshape(k * 64 * 16, 48) for kh in range(5)]
    p = jnp.concatenate(parts, axis=-1)
    acc = jnp.dot(p, w_ref[...], preferred_element_type=jnp.float32)
    y = jnp.maximum(acc + b_ref[...], 0.0)
    o_ref[...] = y.reshape(k, 64, 16, -1).astype(o_ref.dtype)


def _fold_patches(x, Ho, Wo, C):
    k = x.shape[0]
    parts = []
    for kh in range(5):
        a0, p0 = divmod(kh, 2)
        xk = x[:, a0:a0 + Ho, p0]
        for bw, units in _FOLD_TAPS:
            cw = units * C
            parts.append(xk[:, :, bw:bw + Wo, :cw].reshape(k * Ho * Wo, cw))
    return jnp.concatenate(parts, axis=-1)


def _conv_fold_kernel(Ho, Wo, C, x_ref, w_ref, b_ref, o_ref):
    x = x_ref[...]
    k = x.shape[0]
    p = _fold_patches(x, Ho, Wo, C)
    acc = jnp.dot(p, w_ref[...], preferred_element_type=jnp.float32)
    y = jnp.maximum(acc + b_ref[...], 0.0)
    o_ref[...] = y.reshape(k, Ho, Wo, -1).astype(o_ref.dtype)


def _conv_head_kernel(Ho, Wo, C, x_ref, w_ref, b_ref, w5_ref, b5_ref, o_ref):
    x = x_ref[...]
    k = x.shape[0]
    p = _fold_patches(x, Ho, Wo, C)
    acc = jnp.dot(p, w_ref[...], preferred_element_type=jnp.float32)
    h = jnp.maximum(acc + b_ref[...], 0.0)
    hb = h.reshape(k, Ho * Wo, -1)
    logits = jnp.sum(hb * w5_ref[...][None], axis=(1, 2)) + b5_ref[0, 0]
    o_ref[...] = jax.nn.sigmoid(logits).reshape(1, k, 1)


def _params(vmem_mb):
    return pltpu.CompilerParams(
        dimension_semantics=("parallel",),
        vmem_limit_bytes=vmem_mb << 20,
    )


def _conv1_win(xv, w_mat, bias, k):
    B = xv.shape[0]
    N = w_mat.shape[1]
    return pl.pallas_call(
        _conv1_win_kernel,
        out_shape=jax.ShapeDtypeStruct((B, 64, 16, N), jnp.bfloat16),
        grid=(B // k,),
        in_specs=[
            pl.BlockSpec((k, 5, 64, 16, 48), lambda i: (i, 0, 0, 0, 0)),
            pl.BlockSpec(w_mat.shape, lambda i: (0, 0)),
            pl.BlockSpec((1, N), lambda i: (0, 0)),
        ],
        out_specs=pl.BlockSpec((k, 64, 16, N), lambda i: (i, 0, 0, 0)),
        compiler_params=_params(32),
    )(xv, w_mat, bias)


def _fold_dims(xs):
    B, Ha, _, Wf, C2 = xs.shape
    return B, Ha - 2, Wf - 2, C2 // 2


def _conv_fold(xs, w_mat, bias, k):
    B, Ho, Wo, C = _fold_dims(xs)
    K, N = w_mat.shape
    return pl.pallas_call(
        functools.partial(_conv_fold_kernel, Ho, Wo, C),
        out_shape=jax.ShapeDtypeStruct((B, Ho, Wo, N), jnp.bfloat16),
        grid=(B // k,),
        in_specs=[
            pl.BlockSpec((k,) + xs.shape[1:], lambda i: (i, 0, 0, 0, 0)),
            pl.BlockSpec((K, N), lambda i: (0, 0)),
            pl.BlockSpec((1, N), lambda i: (0, 0)),
        ],
        out_specs=pl.BlockSpec((k, Ho, Wo, N), lambda i: (i, 0, 0, 0)),
        compiler_params=_params(52),
    )(xs, w_mat, bias)


def _conv_head(xs, w_mat, bias, w5_mat, b5, k):
    B, Ho, Wo, C = _fold_dims(xs)
    K, N = w_mat.shape
    out = pl.pallas_call(
        functools.partial(_conv_head_kernel, Ho, Wo, C),
        out_shape=jax.ShapeDtypeStruct((B // k, k, 1), jnp.float32),
        grid=(B // k,),
        in_specs=[
            pl.BlockSpec((k,) + xs.shape[1:], lambda i: (i, 0, 0, 0, 0)),
            pl.BlockSpec((K, N), lambda i: (0, 0)),
            pl.BlockSpec((1, N), lambda i: (0, 0)),
            pl.BlockSpec(w5_mat.shape, lambda i: (0, 0)),
            pl.BlockSpec((1, 1), lambda i: (0, 0)),
        ],
        out_specs=pl.BlockSpec((1, k, 1), lambda i: (i, 0, 0)),
        compiler_params=_params(44),
    )(xs, w_mat, bias, w5_mat, b5)
    return out.reshape(B, 1)


def kernel(x, w1, b1, w2, b2, w3, b3, w4, b4, w5, b5):
    B = x.shape[0]
    H = x.shape[2]
    Ho = H // 2

    xh = jnp.transpose(x, (0, 2, 3, 1))
    xf = jnp.pad(xh, ((0, 0), (2, 2), (2, 6), (0, 0))).astype(jnp.bfloat16)
    xfg = xf.reshape(B, H + 4, 17, 24)
    pc = jnp.concatenate([xfg[:, :, :16, :], xfg[:, :, 1:, :]], axis=-1)
    xv = jnp.stack([pc[:, kh:kh + 2 * Ho:2] for kh in range(5)], axis=1)

    n1 = w1.shape[3]
    sel = np.zeros((2, 8, 4, 5), np.float32)
    for s in range(4):
        for kw in range(5):
            g, wp = divmod(2 * s + kw, 8)
            sel[g, wp, s, kw] = 1.0
    w1q = jnp.einsum("gwsk,hkcn->hgwcsn", jnp.asarray(sel), w1)
    w1m = w1q.reshape(240, 4 * n1).astype(jnp.bfloat16)
    b1t = jnp.tile(b1, (1, 4))
    a1 = _conv1_win(xv, w1m, b1t, k=8)
    h = a1.reshape(B, Ho, Ho, n1)

    h = _conv_fold(_pad_fold(h), _pack_weight_fold(w2), b2, k=8)
    h = _conv_fold(_pad_fold(h), _pack_weight_fold(w3), b3, k=8)

    w5m = w5[:, 0].reshape(64, -1)
    return _conv_head(_pad_fold(h), _pack_weight_fold(w4), b4, w5m, b5, k=8)
